# baseline (speedup 1.0000x reference)
import functools
import math

import jax
import jax.numpy as jnp
from jax import lax
from jax.experimental import pallas as pl
from jax.experimental.pallas import tpu as pltpu


def _cdiv(a, b):
    return -(-a // b)


def _gather_kernel(idxt_ref, wblk_ref, out_ref, *, num_categories, chunks):
    c = num_categories
    w = wblk_ref[...]
    idxt = jnp.clip(idxt_ref[...], 0, c - 1)
    iota_c = lax.broadcasted_iota(jnp.int32, (128, c), 1)
    for s in range(chunks):
        t0 = idxt[0:128, s:s + 1]
        t1 = idxt[128:256, s:s + 1]
        onehot = jnp.concatenate(
            [(iota_c == t0).astype(jnp.bfloat16),
             (iota_c == t1).astype(jnp.bfloat16)], axis=1)
        out_ref[pl.ds(128 * s, 128), :] = jax.lax.dot_general(
            onehot, w,
            dimension_numbers=(((1,), (0,)), ((), ())),
            preferred_element_type=jnp.float32,
        )


def kernel(category_indices, weight):
    C, D = weight.shape
    orig_shape = category_indices.shape

    idx = category_indices.reshape(-1).astype(jnp.int32)
    N = idx.shape[0]

    G = 128 // math.gcd(128, D)
    GD = G * D
    chunks = 64
    ptile = 128 * chunks

    P = _cdiv(N, G)
    n_tiles = _cdiv(P, ptile)
    Npad = n_tiles * ptile * G
    if Npad != N:
        idx = jnp.pad(idx, (0, Npad - N))

    idx_t = (idx.reshape(n_tiles, chunks, 128, G)
                .transpose(0, 3, 2, 1)
                .reshape(n_tiles * G * 128, chunks))

    w16 = weight.astype(jnp.bfloat16)
    w_blk = jnp.zeros((G * C, GD), dtype=jnp.bfloat16)
    for g in range(G):
        w_blk = w_blk.at[g * C:(g + 1) * C, g * D:(g + 1) * D].set(w16)

    out_packed = pl.pallas_call(
        functools.partial(_gather_kernel, num_categories=C, chunks=chunks),
        out_shape=jax.ShapeDtypeStruct((P, GD), jnp.float32),
        grid=(n_tiles,),
        in_specs=[
            pl.BlockSpec((G * 128, chunks), lambda i: (i, 0)),
            pl.BlockSpec((G * C, GD), lambda i: (0, 0)),
        ],
        out_specs=pl.BlockSpec((ptile, GD), lambda i: (i, 0)),
        compiler_params=pltpu.CompilerParams(
            dimension_semantics=("parallel",),
        ),
    )(idx_t, w_blk)

    out = out_packed.reshape(P * G, D)[:N]
    return out.reshape(*orig_shape, D)


# speedup vs baseline: 1.9785x; 1.9785x over previous
import functools
import math

import jax
import jax.numpy as jnp
from jax import lax
from jax.experimental import pallas as pl
from jax.experimental.pallas import tpu as pltpu


def _cdiv(a, b):
    return -(-a // b)


def _gather_kernel(idx_ref, w_ref, out_ref, *, num_categories, chunks):
    c = num_categories
    w = w_ref[...]
    idxt = jnp.clip(idx_ref[...], 0, c - 1).T
    iota_c = lax.broadcasted_iota(jnp.int32, (128, c), 1)
    for s in range(chunks):
        onehot = (iota_c == idxt[:, s:s + 1]).astype(jnp.bfloat16)
        out_ref[pl.ds(128 * s, 128), :] = jax.lax.dot_general(
            onehot, w,
            dimension_numbers=(((1,), (0,)), ((), ())),
            preferred_element_type=jnp.float32,
        )


def kernel(category_indices, weight):
    C, D = weight.shape
    orig_shape = category_indices.shape

    idx = category_indices.reshape(-1).astype(jnp.int32)
    N = idx.shape[0]

    chunks = 128
    tile = 128 * chunks
    n_tiles = _cdiv(N, tile)
    Npad = n_tiles * tile
    if Npad != N:
        idx = jnp.pad(idx, (0, Npad - N))

    idx_t = idx.reshape(n_tiles * chunks, 128)

    w16 = weight.astype(jnp.bfloat16)

    out = pl.pallas_call(
        functools.partial(_gather_kernel, num_categories=C, chunks=chunks),
        out_shape=jax.ShapeDtypeStruct((N, D), jnp.float32),
        grid=(n_tiles,),
        in_specs=[
            pl.BlockSpec((chunks, 128), lambda i: (i, 0)),
            pl.BlockSpec((C, D), lambda i: (0, 0)),
        ],
        out_specs=pl.BlockSpec((tile, D), lambda i: (i, 0)),
        compiler_params=pltpu.CompilerParams(
            dimension_semantics=("parallel",),
        ),
    )(idx_t, w16)

    return out.reshape(*orig_shape, D)


# speedup vs baseline: 8.1946x; 4.1417x over previous
import functools

import jax
import jax.numpy as jnp
from jax import lax
from jax.experimental import pallas as pl
from jax.experimental.pallas import tpu as pltpu


def _cdiv(a, b):
    return -(-a // b)


def _gather_kernel(idx_ref, wt_ref, out_ref, *, num_categories, chunks):
    c = num_categories
    wt = wt_ref[...]
    idx = jnp.clip(idx_ref[...], 0, c - 1)
    iota_c = lax.broadcasted_iota(jnp.int32, (c, 128), 0)
    for j in range(chunks):
        row = idx[j:j + 1, :]
        onehot_t = (iota_c == row).astype(jnp.bfloat16)
        out_ref[:, pl.ds(128 * j, 128)] = jax.lax.dot_general(
            wt, onehot_t,
            dimension_numbers=(((1,), (0,)), ((), ())),
            preferred_element_type=jnp.float32,
        )


def kernel(category_indices, weight):
    C, D = weight.shape
    orig_shape = category_indices.shape

    idx = category_indices.reshape(-1).astype(jnp.int32)
    N = idx.shape[0]

    chunks = 128
    tile = 128 * chunks
    n_tiles = _cdiv(N, tile)
    Npad = n_tiles * tile
    if Npad != N:
        idx = jnp.pad(idx, (0, Npad - N))

    idx_2d = idx.reshape(n_tiles * chunks, 128)
    w_t = weight.T.astype(jnp.bfloat16)

    out_t = pl.pallas_call(
        functools.partial(_gather_kernel, num_categories=C, chunks=chunks),
        out_shape=jax.ShapeDtypeStruct((D, N), jnp.float32),
        grid=(n_tiles,),
        in_specs=[
            pl.BlockSpec((chunks, 128), lambda i: (i, 0)),
            pl.BlockSpec((D, C), lambda i: (0, 0)),
        ],
        out_specs=pl.BlockSpec((D, tile), lambda i: (0, i)),
        compiler_params=pltpu.CompilerParams(
            dimension_semantics=("parallel",),
        ),
    )(idx_2d, w_t)

    out = jnp.transpose(out_t)
    return out.reshape(*orig_shape, D)
